# Initial kernel scaffold; baseline (speedup 1.0000x reference)
import jax
import jax.numpy as jnp
from jax.experimental import pallas as pl


def kernel(tokens, embedding):
    raise NotImplementedError("write your pallas kernel here")


# speedup vs baseline: 5.4299x; 5.4299x over previous
import functools
import math

import jax
import jax.numpy as jnp
from jax import lax
from jax.experimental import pallas as pl
from jax.experimental.pallas import tpu as pltpu
from jax.experimental.pallas import tpu_sc as plsc

D_MODEL = 128
SCALE = math.sqrt(float(D_MODEL))

NUM_CORES = 2
NUM_SUBCORES = 16
NW = NUM_CORES * NUM_SUBCORES

IDXW = 128
K = 2
CHUNK = K * IDXW


def _make_gather(vocab: int, batch: int):
    assert batch % (NW * CHUNK) == 0
    rows_per_w = batch // NW
    n_chunks = rows_per_w // CHUNK
    idx_rows_per_w = rows_per_w // IDXW

    mesh = plsc.VectorSubcoreMesh(
        core_axis_name="c", subcore_axis_name="s",
        num_cores=NUM_CORES, num_subcores=NUM_SUBCORES,
    )

    @functools.partial(
        pl.kernel,
        out_type=jax.ShapeDtypeStruct((batch, D_MODEL), jnp.float32),
        mesh=mesh,
        scratch_types=[
            pltpu.VMEM((K, IDXW), jnp.int32),
            pltpu.VMEM((CHUNK, D_MODEL), jnp.float32),
            pltpu.SemaphoreType.DMA,
        ],
    )
    def gather_kernel(table_hbm, idx_hbm, out_hbm, idx_v, rows_v, sem):
        wid = lax.axis_index("s") * NUM_CORES + lax.axis_index("c")
        idx_row0 = wid * idx_rows_per_w
        out_row0 = wid * rows_per_w

        @pl.loop(0, n_chunks)
        def _chunk(g):
            pltpu.sync_copy(idx_hbm.at[pl.ds(idx_row0 + g * K, K)], idx_v)
            copies = [
                pltpu.async_copy(
                    table_hbm.at[idx_v.at[j]],
                    rows_v.at[pl.ds(j * IDXW, IDXW)],
                    sem,
                )
                for j in range(K)
            ]
            for cp in copies:
                cp.wait()

            @pl.loop(0, CHUNK)
            def _scale(i):
                for c in range(D_MODEL // 16):
                    rows_v[i, pl.ds(c * 16, 16)] = (
                        rows_v[i, pl.ds(c * 16, 16)] * SCALE
                    )

            pltpu.sync_copy(rows_v, out_hbm.at[pl.ds(out_row0 + g * CHUNK, CHUNK)])

    return gather_kernel


def kernel(tokens, embedding):
    b, h = tokens.shape
    batch = b * h
    idx2d = tokens.reshape(batch // IDXW, IDXW).astype(jnp.int32)
    out = _make_gather(embedding.shape[0], batch)(embedding, idx2d)
    return out.reshape(b, h, D_MODEL)


# speedup vs baseline: 9.2245x; 1.6988x over previous
import functools
import math

import jax
import jax.numpy as jnp
from jax import lax
from jax.experimental import pallas as pl
from jax.experimental.pallas import tpu as pltpu
from jax.experimental.pallas import tpu_sc as plsc

D_MODEL = 128
SCALE = math.sqrt(float(D_MODEL))

NUM_CORES = 2
NUM_SUBCORES = 16
NW = NUM_CORES * NUM_SUBCORES

CHUNK = 128
NB = 2


def _make_gather(vocab: int, batch: int):
    assert batch % (NW * CHUNK * NB) == 0
    rows_per_w = batch // NW
    n_chunks = rows_per_w // CHUNK
    n_rings = n_chunks // NB

    mesh = plsc.VectorSubcoreMesh(
        core_axis_name="c", subcore_axis_name="s",
        num_cores=NUM_CORES, num_subcores=NUM_SUBCORES,
    )

    @functools.partial(
        pl.kernel,
        out_type=jax.ShapeDtypeStruct((batch, D_MODEL), jnp.float32),
        mesh=mesh,
        scratch_types=[
            pltpu.VMEM((n_chunks, CHUNK), jnp.int32),
            [pltpu.VMEM((CHUNK, D_MODEL), jnp.float32) for _ in range(NB)],
            [pltpu.VMEM((CHUNK, D_MODEL), jnp.float32) for _ in range(NB)],
            [pltpu.SemaphoreType.DMA for _ in range(NB)],
            [pltpu.SemaphoreType.DMA for _ in range(NB)],
        ],
    )
    def gather_kernel(table_hbm, idx_hbm, out_hbm, idx_v, gbufs, wbufs,
                      gsems, osems):
        wid = lax.axis_index("s") * NUM_CORES + lax.axis_index("c")
        out_row0 = wid * rows_per_w

        pltpu.sync_copy(idx_hbm.at[pl.ds(wid * n_chunks, n_chunks)], idx_v)

        def gather_chunk(g, b):
            return pltpu.async_copy(
                table_hbm.at[idx_v.at[g]], gbufs[b], gsems[b])

        def write_chunk(g, b):
            return pltpu.async_copy(
                wbufs[b], out_hbm.at[pl.ds(out_row0 + g * CHUNK, CHUNK)],
                osems[b])

        for b in range(NB):
            gather_chunk(b, b)

        @pl.loop(0, n_rings)
        def _ring(it):
            for b in range(NB):
                g = it * NB + b
                pltpu.make_async_copy(
                    table_hbm.at[idx_v.at[g]], gbufs[b], gsems[b]).wait()
                @pl.when(it > 0)
                def _():
                    pltpu.make_async_copy(
                        wbufs[b],
                        out_hbm.at[pl.ds(out_row0 + (g - NB) * CHUNK, CHUNK)],
                        osems[b]).wait()

                @pl.loop(0, CHUNK)
                def _scale(i):
                    for c in range(D_MODEL // 16):
                        wbufs[b][i, pl.ds(c * 16, 16)] = (
                            gbufs[b][i, pl.ds(c * 16, 16)] * SCALE
                        )

                write_chunk(g, b)

                @pl.when(it < n_rings - 1)
                def _():
                    gather_chunk(g + NB, b)

        for b in range(NB):
            g = n_chunks - NB + b
            pltpu.make_async_copy(
                wbufs[b], out_hbm.at[pl.ds(out_row0 + g * CHUNK, CHUNK)],
                osems[b]).wait()

    return gather_kernel


def kernel(tokens, embedding):
    b, h = tokens.shape
    batch = b * h
    idx2d = tokens.reshape(batch // CHUNK, CHUNK).astype(jnp.int32)
    out = _make_gather(embedding.shape[0], batch)(embedding, idx2d)
    return out.reshape(b, h, D_MODEL)


# speedup vs baseline: 9.2659x; 1.0045x over previous
import functools
import math

import jax
import jax.numpy as jnp
from jax import lax
from jax.experimental import pallas as pl
from jax.experimental.pallas import tpu as pltpu
from jax.experimental.pallas import tpu_sc as plsc

D_MODEL = 128
SCALE = math.sqrt(float(D_MODEL))

NUM_CORES = 2
NUM_SUBCORES = 16
NW = NUM_CORES * NUM_SUBCORES

CHUNK = 128
NB = 2


def _make_gather(vocab: int, batch: int):
    assert batch % (NW * CHUNK * NB) == 0
    rows_per_w = batch // NW
    n_chunks = rows_per_w // CHUNK
    n_rings = n_chunks // NB

    mesh = plsc.VectorSubcoreMesh(
        core_axis_name="c", subcore_axis_name="s",
        num_cores=NUM_CORES, num_subcores=NUM_SUBCORES,
    )

    @functools.partial(
        pl.kernel,
        out_type=jax.ShapeDtypeStruct((batch, D_MODEL), jnp.float32),
        mesh=mesh,
        scratch_types=[
            pltpu.VMEM((n_chunks, CHUNK), jnp.int32),
            [pltpu.VMEM((CHUNK, D_MODEL), jnp.float32) for _ in range(NB)],
            [pltpu.VMEM((CHUNK, D_MODEL), jnp.float32) for _ in range(NB)],
            [pltpu.SemaphoreType.DMA for _ in range(NB)],
            [pltpu.SemaphoreType.DMA for _ in range(NB)],
        ],
    )
    def gather_kernel(table_hbm, idx_hbm, out_hbm, idx_v, gbufs, wbufs,
                      gsems, osems):
        wid = lax.axis_index("s") * NUM_CORES + lax.axis_index("c")
        out_row0 = wid * rows_per_w

        pltpu.sync_copy(idx_hbm.at[pl.ds(wid * n_chunks, n_chunks)], idx_v)

        def gather_chunk(g, b):
            return pltpu.async_copy(
                table_hbm.at[idx_v.at[g]], gbufs[b], gsems[b])

        def write_chunk(g, b):
            return pltpu.async_copy(
                wbufs[b], out_hbm.at[pl.ds(out_row0 + g * CHUNK, CHUNK)],
                osems[b])

        for b in range(NB):
            gather_chunk(b, b)

        @pl.loop(0, n_rings)
        def _ring(it):
            for b in range(NB):
                g = it * NB + b
                pltpu.make_async_copy(
                    table_hbm.at[idx_v.at[g]], gbufs[b], gsems[b]).wait()
                @pl.when(it > 0)
                def _():
                    pltpu.make_async_copy(
                        wbufs[b],
                        out_hbm.at[pl.ds(out_row0 + (g - NB) * CHUNK, CHUNK)],
                        osems[b]).wait()

                @plsc.parallel_loop(0, CHUNK, unroll=4)
                def _scale(i):
                    for c in range(D_MODEL // 16):
                        wbufs[b][i, pl.ds(c * 16, 16)] = (
                            gbufs[b][i, pl.ds(c * 16, 16)] * SCALE
                        )

                write_chunk(g, b)

                @pl.when(it < n_rings - 1)
                def _():
                    gather_chunk(g + NB, b)

        for b in range(NB):
            g = n_chunks - NB + b
            pltpu.make_async_copy(
                wbufs[b], out_hbm.at[pl.ds(out_row0 + g * CHUNK, CHUNK)],
                osems[b]).wait()

    return gather_kernel


def kernel(tokens, embedding):
    b, h = tokens.shape
    batch = b * h
    idx2d = tokens.reshape(batch // CHUNK, CHUNK).astype(jnp.int32)
    out = _make_gather(embedding.shape[0], batch)(embedding, idx2d)
    return out.reshape(b, h, D_MODEL)
